# baseline (speedup 1.0000x reference)
import dataclasses
import functools

import jax
import jax.numpy as jnp
from jax import lax
from jax.experimental import pallas as pl
from jax.experimental.pallas import tpu as pltpu
from jax.experimental.pallas import tpu_sc as plsc

N = 10000
D = 256
DH = D // 2
EPS = 1e-5

NC = 2
NS = 16
NW = NC * NS
NH = N // 2

K = 32
K2 = 64
NBUF = 4
CAPC = 336
CAP_S = CAPC * K
CAP_D = CAPC * K2
CB = 24

ACC_ROWS = 2 * 5056
ROWS_PER_TILE = ACC_ROWS // NS
ROW_BLOCKS = ROWS_PER_TILE // K2
ROW_REM = ROWS_PER_TILE - ROW_BLOCKS * K2
DUMMY = NH + 8

BM = 1000


def _build_contribs(edges):
    e = edges.shape[0]
    i = edges[:, 0]
    j = edges[:, 1]
    dst = jnp.concatenate([i, j])
    src = jnp.concatenate([j, i])
    total = 2 * e
    pt = -(-total // (NW * 16)) * 16
    pad = NW * pt - total
    dst = jnp.concatenate([dst, jnp.full((pad,), NH + DUMMY, jnp.int32)])
    src = jnp.concatenate([src, jnp.zeros((pad,), jnp.int32)])
    return src.reshape(NW, pt), dst.reshape(NW, pt)


def _sc_partition(src_all, dst_all):
    pt = src_all.shape[1]
    mesh = plsc.VectorSubcoreMesh(core_axis_name="c", subcore_axis_name="s")

    @functools.partial(
        pl.kernel,
        out_type=[
            jax.ShapeDtypeStruct((NW, CAP_S), jnp.int32),
            jax.ShapeDtypeStruct((NW, CAP_D), jnp.int32),
            jax.ShapeDtypeStruct((NW, 16), jnp.int32),
            jax.ShapeDtypeStruct((NW, CAP_S), jnp.int32),
            jax.ShapeDtypeStruct((NW, CAP_D), jnp.int32),
            jax.ShapeDtypeStruct((NW, 16), jnp.int32),
        ],
        mesh=mesh,
        scratch_types=[
            pltpu.VMEM((pt,), jnp.int32),
            pltpu.VMEM((pt,), jnp.int32),
            pltpu.VMEM((CAP_S,), jnp.int32),
            pltpu.VMEM((CAP_D,), jnp.int32),
            pltpu.VMEM((CAP_S,), jnp.int32),
            pltpu.VMEM((CAP_D,), jnp.int32),
            pltpu.VMEM((16,), jnp.int32),
            pltpu.VMEM((16,), jnp.int32),
        ],
        compiler_params=dataclasses.replace(
            pltpu.CompilerParams(), needs_layout_passes=False),
    )
    def k(src_hbm, dst_hbm, lsA_hbm, ldA_hbm, cA_hbm, lsB_hbm, ldB_hbm,
          cB_hbm, src_v, dst_v, las, lad, lbs, lbd, ca_v, cb_v):
        cid = lax.axis_index("c")
        sid = lax.axis_index("s")
        p = sid * NC + cid

        pltpu.sync_copy(src_hbm.at[p], src_v)
        pltpu.sync_copy(dst_hbm.at[p], dst_v)

        zero16 = jnp.zeros((16,), jnp.int32)
        dum16 = jnp.full((16,), 2 * DUMMY, jnp.int32)

        @pl.loop(0, CAP_S, step=16)
        def _(o):
            las[pl.ds(o, 16)] = zero16
            lbs[pl.ds(o, 16)] = zero16

        @pl.loop(0, CAP_D, step=16)
        def _(o):
            lad[pl.ds(o, 16)] = dum16
            lbd[pl.ds(o, 16)] = dum16

        @pl.loop(0, pt, step=16, init_carry=(jnp.int32(0), jnp.int32(0)))
        def offs(v, carry):
            off_a, off_b = carry
            sv = src_v[pl.ds(v, 16)]
            dv = dst_v[pl.ds(v, 16)]
            m_a = dv < NH
            ma_i = m_a.astype(jnp.int32)
            cum_a = plsc.cumsum(ma_i)
            n_a = jnp.sum(ma_i)
            r_a = off_a + (cum_a - ma_i)
            plsc.store_scatter(las, [r_a], sv, mask=m_a)
            d2a = dv * 2
            plsc.store_scatter(lad, [2 * r_a], d2a, mask=m_a)
            plsc.store_scatter(lad, [2 * r_a + 1], d2a + 1, mask=m_a)

            m_b = jnp.logical_not(m_a)
            mb_i = m_b.astype(jnp.int32)
            cum_b = plsc.cumsum(mb_i)
            r_b = off_b + (cum_b - mb_i)
            plsc.store_scatter(lbs, [r_b], sv, mask=m_b)
            d2b = (dv - NH) * 2
            plsc.store_scatter(lbd, [2 * r_b], d2b, mask=m_b)
            plsc.store_scatter(lbd, [2 * r_b + 1], d2b + 1, mask=m_b)
            return off_a + n_a, off_b + (16 - n_a)

        off_a, off_b = offs
        ca_v[...] = jnp.full((16,), 1, jnp.int32) * (
            (off_a + NBUF * K - 1) // (NBUF * K) * NBUF)
        cb_v[...] = jnp.full((16,), 1, jnp.int32) * (
            (off_b + NBUF * K - 1) // (NBUF * K) * NBUF)

        pltpu.sync_copy(las, lsA_hbm.at[p])
        pltpu.sync_copy(lad, ldA_hbm.at[p])
        pltpu.sync_copy(lbs, lsB_hbm.at[p])
        pltpu.sync_copy(lbd, ldB_hbm.at[p])
        pltpu.sync_copy(ca_v, cA_hbm.at[p])
        pltpu.sync_copy(cb_v, cB_hbm.at[p])

    return k(src_all, dst_all)


def _sc_aggregate(nbr2, lsA, ldA, cA, lsB, ldB, cB):
    mesh = plsc.VectorSubcoreMesh(core_axis_name="c", subcore_axis_name="s")
    lsA3 = lsA.reshape(NW, CAPC, K)
    ldA3 = ldA.reshape(NW, CAPC, K2)
    lsB3 = lsB.reshape(NW, CAPC, K)
    ldB3 = ldB.reshape(NW, CAPC, K2)

    @functools.partial(
        pl.kernel,
        out_type=jax.ShapeDtypeStruct((NC, ACC_ROWS, DH), jnp.float32),
        mesh=mesh,
        scratch_types=(
            [pltpu.VMEM_SHARED((ACC_ROWS, DH), jnp.float32)]
            + [pltpu.VMEM((CB, K), jnp.int32)]
            + [pltpu.VMEM((CB, K2), jnp.int32)]
            + [pltpu.VMEM((16,), jnp.int32)]
            + [pltpu.VMEM((K2, DH), jnp.float32)] * NBUF
            + [pltpu.SemaphoreType.DMA] * (2 * NBUF)
        ),
    )
    def k(nbr_hbm, lsA_hbm, ldA_hbm, cA_hbm, lsB_hbm, ldB_hbm, cB_hbm,
          agg_hbm, acc, isrc_v, idst_v, cnt_v, *bufs_and_sems):
        gbuf = bufs_and_sems[:NBUF]
        gsem = bufs_and_sems[NBUF:2 * NBUF]
        ssem = bufs_and_sems[2 * NBUF:]
        cid = lax.axis_index("c")
        sid = lax.axis_index("s")

        zf = jnp.zeros((16,), jnp.float32)

        @pl.loop(0, K2)
        def _(r):
            @pl.loop(0, DH, step=16)
            def _(c0):
                gbuf[0][r, pl.ds(c0, 16)] = zf

        @pl.loop(0, ROW_BLOCKS)
        def _(b):
            pltpu.sync_copy(
                gbuf[0], acc.at[pl.ds(sid * ROWS_PER_TILE + b * K2, K2)])

        pltpu.sync_copy(
            gbuf[0].at[pl.ds(0, ROW_REM)],
            acc.at[pl.ds(sid * ROWS_PER_TILE + ROW_BLOCKS * K2, ROW_REM)])

        plsc.subcore_barrier()

        def run_list(ls_hbm, ld_hbm, c_hbm, p):
            pltpu.sync_copy(c_hbm.at[p], cnt_v)
            nch = cnt_v[pl.ds(0, 16)][0]
            nsuper = (nch + CB - 1) // CB

            @pl.loop(0, nsuper)
            def _(s):
                c0 = s * CB
                pltpu.sync_copy(ls_hbm.at[p, pl.ds(c0, CB)], isrc_v)
                pltpu.sync_copy(ld_hbm.at[p, pl.ds(c0, CB)], idst_v)
                nrem = jnp.minimum(nch - c0, CB)

                for b in range(NBUF):
                    pltpu.async_copy(
                        nbr_hbm.at[isrc_v.at[b]],
                        gbuf[b].reshape(K, D), gsem[b])

                @pl.loop(0, nrem, step=NBUF)
                def _(cc):
                    for b in range(NBUF):
                        pltpu.make_async_copy(
                            nbr_hbm.at[isrc_v.at[cc + b]],
                            gbuf[b].reshape(K, D), gsem[b]).wait()
                        pltpu.async_copy(
                            gbuf[b], acc.at[idst_v.at[cc + b]],
                            ssem[b], add=True)

                    for b in range(NBUF):
                        @pl.when(cc + NBUF + b < nrem)
                        def _(b=b):
                            pltpu.make_async_copy(
                                gbuf[b], acc.at[idst_v.at[cc + b]],
                                ssem[b]).wait()
                            pltpu.async_copy(
                                nbr_hbm.at[isrc_v.at[cc + NBUF + b]],
                                gbuf[b].reshape(K, D), gsem[b])

                for b in range(NBUF):
                    pltpu.make_async_copy(
                        gbuf[b], acc.at[idst_v.at[b]], ssem[b]).wait()

        for li in range(2):
            p = sid * 2 + li

            @pl.when(cid == 0)
            def _(p=p):
                run_list(lsA_hbm, ldA_hbm, cA_hbm, p)

            @pl.when(cid == 1)
            def _(p=p):
                run_list(lsB_hbm, ldB_hbm, cB_hbm, p)

        plsc.subcore_barrier()

        @pl.loop(0, ROW_BLOCKS)
        def _(b):
            r0 = sid * ROWS_PER_TILE + b * K2
            pltpu.sync_copy(acc.at[pl.ds(r0, K2)],
                            agg_hbm.at[cid, pl.ds(r0, K2)])

        r1 = sid * ROWS_PER_TILE + ROW_BLOCKS * K2
        pltpu.sync_copy(acc.at[pl.ds(r1, ROW_REM)],
                        agg_hbm.at[cid, pl.ds(r1, ROW_REM)])

    return k(nbr2, lsA3, ldA3, cA, lsB3, ldB3, cB)


def _tc_linear(x, w0, b0, w1, b1):
    def body(x_ref, w0_ref, b0_ref, w1_ref, b1_ref, out_ref, nbr_ref):
        xb = x_ref[...]
        out_ref[...] = (
            jnp.dot(xb, w0_ref[...], preferred_element_type=jnp.float32)
            + b0_ref[...]
        )
        nbr_ref[...] = (
            jnp.dot(xb, w1_ref[...], preferred_element_type=jnp.float32)
            + b1_ref[...]
        )

    grid = N // BM
    return pl.pallas_call(
        body,
        grid=(grid,),
        in_specs=[
            pl.BlockSpec((BM, D), lambda i: (i, 0)),
            pl.BlockSpec((D, D), lambda i: (0, 0)),
            pl.BlockSpec((1, D), lambda i: (0, 0)),
            pl.BlockSpec((D, D), lambda i: (0, 0)),
            pl.BlockSpec((1, D), lambda i: (0, 0)),
        ],
        out_specs=[
            pl.BlockSpec((BM, D), lambda i: (i, 0)),
            pl.BlockSpec((BM, D), lambda i: (i, 0)),
        ],
        out_shape=[
            jax.ShapeDtypeStruct((N, D), jnp.float32),
            jax.ShapeDtypeStruct((N, D), jnp.float32),
        ],
    )(x, w0, b0.reshape(1, D), w1, b1.reshape(1, D))


def _tc_combine(out, agg2, g, be, res=None):
    def body(*refs):
        if res is None:
            out_ref, agg_ref, g_ref, be_ref, y_ref = refs
            r = 0.0
        else:
            out_ref, agg_ref, g_ref, be_ref, res_ref, y_ref = refs
            r = res_ref[...]
        y = out_ref[...] + agg_ref[0]
        mu = jnp.mean(y, axis=-1, keepdims=True)
        yc = y - mu
        var = jnp.mean(yc * yc, axis=-1, keepdims=True)
        yn = yc * lax.rsqrt(var + EPS) * g_ref[...] + be_ref[...]
        y_ref[...] = jnp.maximum(yn + r, 0.0)

    grid = N // BM
    npb = NH // BM
    in_specs = [
        pl.BlockSpec((BM, D), lambda i: (i, 0)),
        pl.BlockSpec((1, BM, D), lambda i: (i // npb, i % npb, 0)),
        pl.BlockSpec((1, D), lambda i: (0, 0)),
        pl.BlockSpec((1, D), lambda i: (0, 0)),
    ]
    args = [out, agg2, g.reshape(1, D), be.reshape(1, D)]
    if res is not None:
        in_specs.append(pl.BlockSpec((BM, D), lambda i: (i, 0)))
        args.append(res)
    return pl.pallas_call(
        body,
        grid=(grid,),
        in_specs=in_specs,
        out_specs=pl.BlockSpec((BM, D), lambda i: (i, 0)),
        out_shape=jax.ShapeDtypeStruct((N, D), jnp.float32),
    )(*args)


def kernel(features, edges, w0_f, b0_f, w1_f, b1_f, g_f, be_f,
           w0_h1, b0_h1, w1_h1, b1_h1, g_h1, be_h1,
           w0_h2, b0_h2, w1_h2, b1_h2, g_h2, be_h2):
    src_all, dst_all = _build_contribs(edges)
    lsA, ldA, cA, lsB, ldB, cB = _sc_partition(src_all, dst_all)
    layers = [
        (w0_f, b0_f, w1_f, b1_f, g_f, be_f),
        (w0_h1, b0_h1, w1_h1, b1_h1, g_h1, be_h1),
        (w0_h2, b0_h2, w1_h2, b1_h2, g_h2, be_h2),
    ]
    x = features
    for li, (w0, b0, w1, b1, g, be) in enumerate(layers):
        out, nbr = _tc_linear(x, w0, b0, w1, b1)
        agg2 = _sc_aggregate(nbr, lsA, ldA, cA, lsB, ldB, cB)
        agg = agg2.reshape(NC, ACC_ROWS // 2, D)
        x = _tc_combine(out, agg, g, be, res=features if li == 2 else None)
    return x


# speedup vs baseline: 1.0447x; 1.0447x over previous
import dataclasses
import functools

import jax
import jax.numpy as jnp
from jax import lax
from jax.experimental import pallas as pl
from jax.experimental.pallas import tpu as pltpu
from jax.experimental.pallas import tpu_sc as plsc

N = 10000
D = 256
DH = D // 2
EPS = 1e-5

NC = 2
NS = 16
NW = NC * NS
NH = N // 2

K = 40
K2 = 80
NBUF = 3
CAPC = 288
CAP_S = CAPC * K
CAP_D = CAPC * K2
CB = 48

ACC_ROWS = 2 * 5056
ROWS_PER_TILE = ACC_ROWS // NS
ROW_BLOCKS = ROWS_PER_TILE // K2
ROW_REM = ROWS_PER_TILE - ROW_BLOCKS * K2
DUMMY = NH + 8

BM = 1000


def _build_contribs(edges):
    e = edges.shape[0]
    i = edges[:, 0]
    j = edges[:, 1]
    dst = jnp.concatenate([i, j])
    src = jnp.concatenate([j, i])
    total = 2 * e
    pt = -(-total // (NW * 16)) * 16
    pad = NW * pt - total
    dst = jnp.concatenate([dst, jnp.full((pad,), NH + DUMMY, jnp.int32)])
    src = jnp.concatenate([src, jnp.zeros((pad,), jnp.int32)])
    return src.reshape(NW, pt), dst.reshape(NW, pt)


def _sc_partition(src_all, dst_all):
    pt = src_all.shape[1]
    mesh = plsc.VectorSubcoreMesh(core_axis_name="c", subcore_axis_name="s")

    @functools.partial(
        pl.kernel,
        out_type=[
            jax.ShapeDtypeStruct((NW, CAP_S), jnp.int32),
            jax.ShapeDtypeStruct((NW, CAP_D), jnp.int32),
            jax.ShapeDtypeStruct((NW, 16), jnp.int32),
            jax.ShapeDtypeStruct((NW, CAP_S), jnp.int32),
            jax.ShapeDtypeStruct((NW, CAP_D), jnp.int32),
            jax.ShapeDtypeStruct((NW, 16), jnp.int32),
        ],
        mesh=mesh,
        scratch_types=[
            pltpu.VMEM((pt,), jnp.int32),
            pltpu.VMEM((pt,), jnp.int32),
            pltpu.VMEM((CAP_S,), jnp.int32),
            pltpu.VMEM((CAP_D,), jnp.int32),
            pltpu.VMEM((CAP_S,), jnp.int32),
            pltpu.VMEM((CAP_D,), jnp.int32),
            pltpu.VMEM((16,), jnp.int32),
            pltpu.VMEM((16,), jnp.int32),
        ],
        compiler_params=dataclasses.replace(
            pltpu.CompilerParams(), needs_layout_passes=False),
    )
    def k(src_hbm, dst_hbm, lsA_hbm, ldA_hbm, cA_hbm, lsB_hbm, ldB_hbm,
          cB_hbm, src_v, dst_v, las, lad, lbs, lbd, ca_v, cb_v):
        cid = lax.axis_index("c")
        sid = lax.axis_index("s")
        p = sid * NC + cid

        pltpu.sync_copy(src_hbm.at[p], src_v)
        pltpu.sync_copy(dst_hbm.at[p], dst_v)

        zero16 = jnp.zeros((16,), jnp.int32)
        dum16 = jnp.full((16,), 2 * DUMMY, jnp.int32)

        @pl.loop(0, CAP_S, step=16)
        def _(o):
            las[pl.ds(o, 16)] = zero16
            lbs[pl.ds(o, 16)] = zero16

        @pl.loop(0, CAP_D, step=16)
        def _(o):
            lad[pl.ds(o, 16)] = dum16
            lbd[pl.ds(o, 16)] = dum16

        @pl.loop(0, pt, step=16, init_carry=(jnp.int32(0), jnp.int32(0)))
        def offs(v, carry):
            off_a, off_b = carry
            sv = src_v[pl.ds(v, 16)]
            dv = dst_v[pl.ds(v, 16)]
            m_a = dv < NH
            ma_i = m_a.astype(jnp.int32)
            cum_a = plsc.cumsum(ma_i)
            n_a = jnp.sum(ma_i)
            r_a = off_a + (cum_a - ma_i)
            plsc.store_scatter(las, [r_a], sv, mask=m_a)
            d2a = dv * 2
            plsc.store_scatter(lad, [2 * r_a], d2a, mask=m_a)
            plsc.store_scatter(lad, [2 * r_a + 1], d2a + 1, mask=m_a)

            m_b = jnp.logical_not(m_a)
            mb_i = m_b.astype(jnp.int32)
            cum_b = plsc.cumsum(mb_i)
            r_b = off_b + (cum_b - mb_i)
            plsc.store_scatter(lbs, [r_b], sv, mask=m_b)
            d2b = (dv - NH) * 2
            plsc.store_scatter(lbd, [2 * r_b], d2b, mask=m_b)
            plsc.store_scatter(lbd, [2 * r_b + 1], d2b + 1, mask=m_b)
            return off_a + n_a, off_b + (16 - n_a)

        off_a, off_b = offs
        ca_v[...] = jnp.full((16,), 1, jnp.int32) * (
            (off_a + NBUF * K - 1) // (NBUF * K) * NBUF)
        cb_v[...] = jnp.full((16,), 1, jnp.int32) * (
            (off_b + NBUF * K - 1) // (NBUF * K) * NBUF)

        pltpu.sync_copy(las, lsA_hbm.at[p])
        pltpu.sync_copy(lad, ldA_hbm.at[p])
        pltpu.sync_copy(lbs, lsB_hbm.at[p])
        pltpu.sync_copy(lbd, ldB_hbm.at[p])
        pltpu.sync_copy(ca_v, cA_hbm.at[p])
        pltpu.sync_copy(cb_v, cB_hbm.at[p])

    return k(src_all, dst_all)


def _sc_aggregate(nbr2, lsA, ldA, cA, lsB, ldB, cB):
    mesh = plsc.VectorSubcoreMesh(core_axis_name="c", subcore_axis_name="s")
    lsA3 = lsA.reshape(NW, CAPC, K)
    ldA3 = ldA.reshape(NW, CAPC, K2)
    lsB3 = lsB.reshape(NW, CAPC, K)
    ldB3 = ldB.reshape(NW, CAPC, K2)

    @functools.partial(
        pl.kernel,
        out_type=jax.ShapeDtypeStruct((NC, ACC_ROWS, DH), jnp.float32),
        mesh=mesh,
        scratch_types=(
            [pltpu.VMEM_SHARED((ACC_ROWS, DH), jnp.float32)]
            + [pltpu.VMEM((CB, K), jnp.int32)]
            + [pltpu.VMEM((CB, K2), jnp.int32)]
            + [pltpu.VMEM((16,), jnp.int32)]
            + [pltpu.VMEM((K2, DH), jnp.float32)] * NBUF
            + [pltpu.SemaphoreType.DMA] * (2 * NBUF)
        ),
    )
    def k(nbr_hbm, lsA_hbm, ldA_hbm, cA_hbm, lsB_hbm, ldB_hbm, cB_hbm,
          agg_hbm, acc, isrc_v, idst_v, cnt_v, *bufs_and_sems):
        gbuf = bufs_and_sems[:NBUF]
        gsem = bufs_and_sems[NBUF:2 * NBUF]
        ssem = bufs_and_sems[2 * NBUF:]
        cid = lax.axis_index("c")
        sid = lax.axis_index("s")

        zf = jnp.zeros((16,), jnp.float32)

        @pl.loop(0, K2)
        def _(r):
            @pl.loop(0, DH, step=16)
            def _(c0):
                gbuf[0][r, pl.ds(c0, 16)] = zf

        @pl.loop(0, ROW_BLOCKS)
        def _(b):
            pltpu.sync_copy(
                gbuf[0], acc.at[pl.ds(sid * ROWS_PER_TILE + b * K2, K2)])

        pltpu.sync_copy(
            gbuf[0].at[pl.ds(0, ROW_REM)],
            acc.at[pl.ds(sid * ROWS_PER_TILE + ROW_BLOCKS * K2, ROW_REM)])

        plsc.subcore_barrier()

        def run_list(ls_hbm, ld_hbm, c_hbm, p):
            pltpu.sync_copy(c_hbm.at[p], cnt_v)
            nch = cnt_v[pl.ds(0, 16)][0]
            nsuper = (nch + CB - 1) // CB

            @pl.loop(0, nsuper)
            def _(s):
                c0 = s * CB
                pltpu.sync_copy(ls_hbm.at[p, pl.ds(c0, CB)], isrc_v)
                pltpu.sync_copy(ld_hbm.at[p, pl.ds(c0, CB)], idst_v)
                nrem = jnp.minimum(nch - c0, CB)

                for b in range(NBUF):
                    pltpu.async_copy(
                        nbr_hbm.at[isrc_v.at[b]],
                        gbuf[b].reshape(K, D), gsem[b])

                @pl.loop(0, nrem, step=NBUF)
                def _(cc):
                    for b in range(NBUF):
                        pltpu.make_async_copy(
                            nbr_hbm.at[isrc_v.at[cc + b]],
                            gbuf[b].reshape(K, D), gsem[b]).wait()
                        pltpu.async_copy(
                            gbuf[b], acc.at[idst_v.at[cc + b]],
                            ssem[b], add=True)

                    for b in range(NBUF):
                        @pl.when(cc + NBUF + b < nrem)
                        def _(b=b):
                            pltpu.make_async_copy(
                                gbuf[b], acc.at[idst_v.at[cc + b]],
                                ssem[b]).wait()
                            pltpu.async_copy(
                                nbr_hbm.at[isrc_v.at[cc + NBUF + b]],
                                gbuf[b].reshape(K, D), gsem[b])

                for b in range(NBUF):
                    pltpu.make_async_copy(
                        gbuf[b], acc.at[idst_v.at[b]], ssem[b]).wait()

        for li in range(2):
            p = sid * 2 + li

            @pl.when(cid == 0)
            def _(p=p):
                run_list(lsA_hbm, ldA_hbm, cA_hbm, p)

            @pl.when(cid == 1)
            def _(p=p):
                run_list(lsB_hbm, ldB_hbm, cB_hbm, p)

        plsc.subcore_barrier()

        @pl.loop(0, ROW_BLOCKS)
        def _(b):
            r0 = sid * ROWS_PER_TILE + b * K2
            pltpu.sync_copy(acc.at[pl.ds(r0, K2)],
                            agg_hbm.at[cid, pl.ds(r0, K2)])

        r1 = sid * ROWS_PER_TILE + ROW_BLOCKS * K2
        pltpu.sync_copy(acc.at[pl.ds(r1, ROW_REM)],
                        agg_hbm.at[cid, pl.ds(r1, ROW_REM)])

    return k(nbr2, lsA3, ldA3, cA, lsB3, ldB3, cB)


def _tc_linear(x, w0, b0, w1, b1):
    def body(x_ref, w0_ref, b0_ref, w1_ref, b1_ref, out_ref, nbr_ref):
        xb = x_ref[...]
        out_ref[...] = (
            jnp.dot(xb, w0_ref[...], preferred_element_type=jnp.float32)
            + b0_ref[...]
        )
        nbr_ref[...] = (
            jnp.dot(xb, w1_ref[...], preferred_element_type=jnp.float32)
            + b1_ref[...]
        )

    grid = N // BM
    return pl.pallas_call(
        body,
        grid=(grid,),
        in_specs=[
            pl.BlockSpec((BM, D), lambda i: (i, 0)),
            pl.BlockSpec((D, D), lambda i: (0, 0)),
            pl.BlockSpec((1, D), lambda i: (0, 0)),
            pl.BlockSpec((D, D), lambda i: (0, 0)),
            pl.BlockSpec((1, D), lambda i: (0, 0)),
        ],
        out_specs=[
            pl.BlockSpec((BM, D), lambda i: (i, 0)),
            pl.BlockSpec((BM, D), lambda i: (i, 0)),
        ],
        out_shape=[
            jax.ShapeDtypeStruct((N, D), jnp.float32),
            jax.ShapeDtypeStruct((N, D), jnp.float32),
        ],
    )(x, w0, b0.reshape(1, D), w1, b1.reshape(1, D))


def _tc_combine(out, agg2, g, be, res=None):
    def body(*refs):
        if res is None:
            out_ref, agg_ref, g_ref, be_ref, y_ref = refs
            r = 0.0
        else:
            out_ref, agg_ref, g_ref, be_ref, res_ref, y_ref = refs
            r = res_ref[...]
        y = out_ref[...] + agg_ref[0]
        mu = jnp.mean(y, axis=-1, keepdims=True)
        yc = y - mu
        var = jnp.mean(yc * yc, axis=-1, keepdims=True)
        yn = yc * lax.rsqrt(var + EPS) * g_ref[...] + be_ref[...]
        y_ref[...] = jnp.maximum(yn + r, 0.0)

    grid = N // BM
    npb = NH // BM
    in_specs = [
        pl.BlockSpec((BM, D), lambda i: (i, 0)),
        pl.BlockSpec((1, BM, D), lambda i: (i // npb, i % npb, 0)),
        pl.BlockSpec((1, D), lambda i: (0, 0)),
        pl.BlockSpec((1, D), lambda i: (0, 0)),
    ]
    args = [out, agg2, g.reshape(1, D), be.reshape(1, D)]
    if res is not None:
        in_specs.append(pl.BlockSpec((BM, D), lambda i: (i, 0)))
        args.append(res)
    return pl.pallas_call(
        body,
        grid=(grid,),
        in_specs=in_specs,
        out_specs=pl.BlockSpec((BM, D), lambda i: (i, 0)),
        out_shape=jax.ShapeDtypeStruct((N, D), jnp.float32),
    )(*args)


def kernel(features, edges, w0_f, b0_f, w1_f, b1_f, g_f, be_f,
           w0_h1, b0_h1, w1_h1, b1_h1, g_h1, be_h1,
           w0_h2, b0_h2, w1_h2, b1_h2, g_h2, be_h2):
    src_all, dst_all = _build_contribs(edges)
    lsA, ldA, cA, lsB, ldB, cB = _sc_partition(src_all, dst_all)
    layers = [
        (w0_f, b0_f, w1_f, b1_f, g_f, be_f),
        (w0_h1, b0_h1, w1_h1, b1_h1, g_h1, be_h1),
        (w0_h2, b0_h2, w1_h2, b1_h2, g_h2, be_h2),
    ]
    x = features
    for li, (w0, b0, w1, b1, g, be) in enumerate(layers):
        out, nbr = _tc_linear(x, w0, b0, w1, b1)
        agg2 = _sc_aggregate(nbr, lsA, ldA, cA, lsB, ldB, cB)
        agg = agg2.reshape(NC, ACC_ROWS // 2, D)
        x = _tc_combine(out, agg, g, be, res=features if li == 2 else None)
    return x


# speedup vs baseline: 1.0538x; 1.0087x over previous
import dataclasses
import functools

import jax
import jax.numpy as jnp
from jax import lax
from jax.experimental import pallas as pl
from jax.experimental.pallas import tpu as pltpu
from jax.experimental.pallas import tpu_sc as plsc

N = 10000
D = 256
DH = D // 2
EPS = 1e-5

NC = 2
NS = 16
NW = NC * NS
NH = N // 2

K = 40
K2 = 80
NBUF = 3
CAPC = 288
CAP_S = CAPC * K
CAP_D = CAPC * K2
CB = 72

ACC_ROWS = 2 * 5056
ROWS_PER_TILE = ACC_ROWS // NS
ROW_BLOCKS = ROWS_PER_TILE // K2
ROW_REM = ROWS_PER_TILE - ROW_BLOCKS * K2
DUMMY = NH + 8

BM = 1000


def _build_contribs(edges):
    e = edges.shape[0]
    i = edges[:, 0]
    j = edges[:, 1]
    dst = jnp.concatenate([i, j])
    src = jnp.concatenate([j, i])
    total = 2 * e
    pt = -(-total // (NW * 16)) * 16
    pad = NW * pt - total
    dst = jnp.concatenate([dst, jnp.full((pad,), NH + DUMMY, jnp.int32)])
    src = jnp.concatenate([src, jnp.zeros((pad,), jnp.int32)])
    return src.reshape(NW, pt), dst.reshape(NW, pt)


def _sc_partition(src_all, dst_all):
    pt = src_all.shape[1]
    mesh = plsc.VectorSubcoreMesh(core_axis_name="c", subcore_axis_name="s")

    @functools.partial(
        pl.kernel,
        out_type=[
            jax.ShapeDtypeStruct((NW, CAP_S), jnp.int32),
            jax.ShapeDtypeStruct((NW, CAP_D), jnp.int32),
            jax.ShapeDtypeStruct((NW, 16), jnp.int32),
            jax.ShapeDtypeStruct((NW, CAP_S), jnp.int32),
            jax.ShapeDtypeStruct((NW, CAP_D), jnp.int32),
            jax.ShapeDtypeStruct((NW, 16), jnp.int32),
        ],
        mesh=mesh,
        scratch_types=[
            pltpu.VMEM((pt,), jnp.int32),
            pltpu.VMEM((pt,), jnp.int32),
            pltpu.VMEM((CAP_S,), jnp.int32),
            pltpu.VMEM((CAP_D,), jnp.int32),
            pltpu.VMEM((CAP_S,), jnp.int32),
            pltpu.VMEM((CAP_D,), jnp.int32),
            pltpu.VMEM((16,), jnp.int32),
            pltpu.VMEM((16,), jnp.int32),
        ],
        compiler_params=dataclasses.replace(
            pltpu.CompilerParams(), needs_layout_passes=False),
    )
    def k(src_hbm, dst_hbm, lsA_hbm, ldA_hbm, cA_hbm, lsB_hbm, ldB_hbm,
          cB_hbm, src_v, dst_v, las, lad, lbs, lbd, ca_v, cb_v):
        cid = lax.axis_index("c")
        sid = lax.axis_index("s")
        p = sid * NC + cid

        pltpu.sync_copy(src_hbm.at[p], src_v)
        pltpu.sync_copy(dst_hbm.at[p], dst_v)

        zero16 = jnp.zeros((16,), jnp.int32)
        dum16 = jnp.full((16,), 2 * DUMMY, jnp.int32)

        @pl.loop(0, CAP_S, step=16)
        def _(o):
            las[pl.ds(o, 16)] = zero16
            lbs[pl.ds(o, 16)] = zero16

        @pl.loop(0, CAP_D, step=16)
        def _(o):
            lad[pl.ds(o, 16)] = dum16
            lbd[pl.ds(o, 16)] = dum16

        @pl.loop(0, pt, step=16, init_carry=(jnp.int32(0), jnp.int32(0)))
        def offs(v, carry):
            off_a, off_b = carry
            sv = src_v[pl.ds(v, 16)]
            dv = dst_v[pl.ds(v, 16)]
            m_a = dv < NH
            ma_i = m_a.astype(jnp.int32)
            cum_a = plsc.cumsum(ma_i)
            n_a = jnp.sum(ma_i)
            r_a = off_a + (cum_a - ma_i)
            plsc.store_scatter(las, [r_a], sv, mask=m_a)
            d2a = dv * 2
            plsc.store_scatter(lad, [2 * r_a], d2a, mask=m_a)
            plsc.store_scatter(lad, [2 * r_a + 1], d2a + 1, mask=m_a)

            m_b = jnp.logical_not(m_a)
            mb_i = m_b.astype(jnp.int32)
            cum_b = plsc.cumsum(mb_i)
            r_b = off_b + (cum_b - mb_i)
            plsc.store_scatter(lbs, [r_b], sv, mask=m_b)
            d2b = (dv - NH) * 2
            plsc.store_scatter(lbd, [2 * r_b], d2b, mask=m_b)
            plsc.store_scatter(lbd, [2 * r_b + 1], d2b + 1, mask=m_b)
            return off_a + n_a, off_b + (16 - n_a)

        off_a, off_b = offs
        ca_v[...] = jnp.full((16,), 1, jnp.int32) * (
            (off_a + NBUF * K - 1) // (NBUF * K) * NBUF)
        cb_v[...] = jnp.full((16,), 1, jnp.int32) * (
            (off_b + NBUF * K - 1) // (NBUF * K) * NBUF)

        pltpu.sync_copy(las, lsA_hbm.at[p])
        pltpu.sync_copy(lad, ldA_hbm.at[p])
        pltpu.sync_copy(lbs, lsB_hbm.at[p])
        pltpu.sync_copy(lbd, ldB_hbm.at[p])
        pltpu.sync_copy(ca_v, cA_hbm.at[p])
        pltpu.sync_copy(cb_v, cB_hbm.at[p])

    return k(src_all, dst_all)


def _sc_aggregate(nbr2, lsA, ldA, cA, lsB, ldB, cB):
    mesh = plsc.VectorSubcoreMesh(core_axis_name="c", subcore_axis_name="s")
    lsA3 = lsA.reshape(NW, CAPC, K)
    ldA3 = ldA.reshape(NW, CAPC, K2)
    lsB3 = lsB.reshape(NW, CAPC, K)
    ldB3 = ldB.reshape(NW, CAPC, K2)

    @functools.partial(
        pl.kernel,
        out_type=jax.ShapeDtypeStruct((NC, ACC_ROWS, DH), jnp.float32),
        mesh=mesh,
        scratch_types=(
            [pltpu.VMEM_SHARED((ACC_ROWS, DH), jnp.float32)]
            + [pltpu.VMEM((CB, K), jnp.int32)]
            + [pltpu.VMEM((CB, K2), jnp.int32)]
            + [pltpu.VMEM((16,), jnp.int32)]
            + [pltpu.VMEM((K2, DH), jnp.float32)] * NBUF
            + [pltpu.SemaphoreType.DMA] * (2 * NBUF)
        ),
    )
    def k(nbr_hbm, lsA_hbm, ldA_hbm, cA_hbm, lsB_hbm, ldB_hbm, cB_hbm,
          agg_hbm, acc, isrc_v, idst_v, cnt_v, *bufs_and_sems):
        gbuf = bufs_and_sems[:NBUF]
        gsem = bufs_and_sems[NBUF:2 * NBUF]
        ssem = bufs_and_sems[2 * NBUF:]
        cid = lax.axis_index("c")
        sid = lax.axis_index("s")

        zf = jnp.zeros((16,), jnp.float32)

        @pl.loop(0, K2)
        def _(r):
            @pl.loop(0, DH, step=16)
            def _(c0):
                gbuf[0][r, pl.ds(c0, 16)] = zf

        @pl.loop(0, ROW_BLOCKS)
        def _(b):
            pltpu.sync_copy(
                gbuf[0], acc.at[pl.ds(sid * ROWS_PER_TILE + b * K2, K2)])

        pltpu.sync_copy(
            gbuf[0].at[pl.ds(0, ROW_REM)],
            acc.at[pl.ds(sid * ROWS_PER_TILE + ROW_BLOCKS * K2, ROW_REM)])

        plsc.subcore_barrier()

        def run_list(ls_hbm, ld_hbm, c_hbm, p):
            pltpu.sync_copy(c_hbm.at[p], cnt_v)
            nch = cnt_v[pl.ds(0, 16)][0]
            nsuper = (nch + CB - 1) // CB

            @pl.loop(0, nsuper)
            def _(s):
                c0 = s * CB
                pltpu.sync_copy(ls_hbm.at[p, pl.ds(c0, CB)], isrc_v)
                pltpu.sync_copy(ld_hbm.at[p, pl.ds(c0, CB)], idst_v)
                nrem = jnp.minimum(nch - c0, CB)

                for b in range(NBUF):
                    pltpu.async_copy(
                        nbr_hbm.at[isrc_v.at[b]],
                        gbuf[b].reshape(K, D), gsem[b])

                @pl.loop(0, nrem, step=NBUF)
                def _(cc):
                    for b in range(NBUF):
                        pltpu.make_async_copy(
                            nbr_hbm.at[isrc_v.at[cc + b]],
                            gbuf[b].reshape(K, D), gsem[b]).wait()
                        pltpu.async_copy(
                            gbuf[b], acc.at[idst_v.at[cc + b]],
                            ssem[b], add=True)

                    for b in range(NBUF):
                        @pl.when(cc + NBUF + b < nrem)
                        def _(b=b):
                            pltpu.make_async_copy(
                                gbuf[b], acc.at[idst_v.at[cc + b]],
                                ssem[b]).wait()
                            pltpu.async_copy(
                                nbr_hbm.at[isrc_v.at[cc + NBUF + b]],
                                gbuf[b].reshape(K, D), gsem[b])

                for b in range(NBUF):
                    pltpu.make_async_copy(
                        gbuf[b], acc.at[idst_v.at[b]], ssem[b]).wait()

        for li in range(2):
            p = sid * 2 + li

            @pl.when(cid == 0)
            def _(p=p):
                run_list(lsA_hbm, ldA_hbm, cA_hbm, p)

            @pl.when(cid == 1)
            def _(p=p):
                run_list(lsB_hbm, ldB_hbm, cB_hbm, p)

        plsc.subcore_barrier()

        @pl.loop(0, ROW_BLOCKS)
        def _(b):
            r0 = sid * ROWS_PER_TILE + b * K2
            pltpu.sync_copy(acc.at[pl.ds(r0, K2)],
                            agg_hbm.at[cid, pl.ds(r0, K2)])

        r1 = sid * ROWS_PER_TILE + ROW_BLOCKS * K2
        pltpu.sync_copy(acc.at[pl.ds(r1, ROW_REM)],
                        agg_hbm.at[cid, pl.ds(r1, ROW_REM)])

    return k(nbr2, lsA3, ldA3, cA, lsB3, ldB3, cB)


def _tc_linear(x, w0, b0, w1, b1):
    def body(x_ref, w0_ref, b0_ref, w1_ref, b1_ref, out_ref, nbr_ref):
        xb = x_ref[...]
        out_ref[...] = (
            jnp.dot(xb, w0_ref[...], preferred_element_type=jnp.float32)
            + b0_ref[...]
        )
        nbr_ref[...] = (
            jnp.dot(xb, w1_ref[...], preferred_element_type=jnp.float32)
            + b1_ref[...]
        )

    grid = N // BM
    return pl.pallas_call(
        body,
        grid=(grid,),
        in_specs=[
            pl.BlockSpec((BM, D), lambda i: (i, 0)),
            pl.BlockSpec((D, D), lambda i: (0, 0)),
            pl.BlockSpec((1, D), lambda i: (0, 0)),
            pl.BlockSpec((D, D), lambda i: (0, 0)),
            pl.BlockSpec((1, D), lambda i: (0, 0)),
        ],
        out_specs=[
            pl.BlockSpec((BM, D), lambda i: (i, 0)),
            pl.BlockSpec((BM, D), lambda i: (i, 0)),
        ],
        out_shape=[
            jax.ShapeDtypeStruct((N, D), jnp.float32),
            jax.ShapeDtypeStruct((N, D), jnp.float32),
        ],
    )(x, w0, b0.reshape(1, D), w1, b1.reshape(1, D))


def _tc_combine(out, agg2, g, be, res=None):
    def body(*refs):
        if res is None:
            out_ref, agg_ref, g_ref, be_ref, y_ref = refs
            r = 0.0
        else:
            out_ref, agg_ref, g_ref, be_ref, res_ref, y_ref = refs
            r = res_ref[...]
        y = out_ref[...] + agg_ref[0]
        mu = jnp.mean(y, axis=-1, keepdims=True)
        yc = y - mu
        var = jnp.mean(yc * yc, axis=-1, keepdims=True)
        yn = yc * lax.rsqrt(var + EPS) * g_ref[...] + be_ref[...]
        y_ref[...] = jnp.maximum(yn + r, 0.0)

    grid = N // BM
    npb = NH // BM
    in_specs = [
        pl.BlockSpec((BM, D), lambda i: (i, 0)),
        pl.BlockSpec((1, BM, D), lambda i: (i // npb, i % npb, 0)),
        pl.BlockSpec((1, D), lambda i: (0, 0)),
        pl.BlockSpec((1, D), lambda i: (0, 0)),
    ]
    args = [out, agg2, g.reshape(1, D), be.reshape(1, D)]
    if res is not None:
        in_specs.append(pl.BlockSpec((BM, D), lambda i: (i, 0)))
        args.append(res)
    return pl.pallas_call(
        body,
        grid=(grid,),
        in_specs=in_specs,
        out_specs=pl.BlockSpec((BM, D), lambda i: (i, 0)),
        out_shape=jax.ShapeDtypeStruct((N, D), jnp.float32),
    )(*args)


def kernel(features, edges, w0_f, b0_f, w1_f, b1_f, g_f, be_f,
           w0_h1, b0_h1, w1_h1, b1_h1, g_h1, be_h1,
           w0_h2, b0_h2, w1_h2, b1_h2, g_h2, be_h2):
    src_all, dst_all = _build_contribs(edges)
    lsA, ldA, cA, lsB, ldB, cB = _sc_partition(src_all, dst_all)
    layers = [
        (w0_f, b0_f, w1_f, b1_f, g_f, be_f),
        (w0_h1, b0_h1, w1_h1, b1_h1, g_h1, be_h1),
        (w0_h2, b0_h2, w1_h2, b1_h2, g_h2, be_h2),
    ]
    x = features
    for li, (w0, b0, w1, b1, g, be) in enumerate(layers):
        out, nbr = _tc_linear(x, w0, b0, w1, b1)
        agg2 = _sc_aggregate(nbr, lsA, ldA, cA, lsB, ldB, cB)
        agg = agg2.reshape(NC, ACC_ROWS // 2, D)
        x = _tc_combine(out, agg, g, be, res=features if li == 2 else None)
    return x
